# baseline (speedup 1.0000x reference)
import functools
import jax
import jax.numpy as jnp
from jax import lax
from jax.experimental import pallas as pl
from jax.experimental.pallas import tpu as pltpu
from jax.experimental.pallas import tpu_sc as plsc

_N = 64
_NQ = 9
_RB = 512
_NW = 32
_ROWS = 1024 * 512
_RPW = _ROWS // _NW
_NBLK = _RPW // _RB


def _sort64(a, b, c, d):
    a = lax.sort(a)
    b = lax.sort(b)
    c = lax.sort(c)
    d = lax.sort(d)
    br = lax.rev(b, (0,))
    a, b = lax.sort(jnp.minimum(a, br)), lax.sort(jnp.maximum(a, br))
    dr = lax.rev(d, (0,))
    c, d = lax.sort(jnp.minimum(c, dr)), lax.sort(jnp.maximum(c, dr))
    dr = lax.rev(d, (0,))
    cr = lax.rev(c, (0,))
    l0, l1 = jnp.minimum(a, dr), jnp.minimum(b, cr)
    h0, h1 = jnp.maximum(a, dr), jnp.maximum(b, cr)
    s0 = lax.sort(jnp.minimum(l0, l1))
    s1 = lax.sort(jnp.maximum(l0, l1))
    s2 = lax.sort(jnp.minimum(h0, h1))
    s3 = lax.sort(jnp.maximum(h0, h1))
    return s0, s1, s2, s3


def _make_kernel():
    mesh = plsc.VectorSubcoreMesh(core_axis_name="c", subcore_axis_name="s")

    @functools.partial(
        pl.kernel,
        mesh=mesh,
        out_type=jax.ShapeDtypeStruct((_ROWS * _NQ,), jnp.float32),
        scratch_types=[
            pltpu.VMEM((_RB * _N,), jnp.float32),
            pltpu.VMEM((_RB * _NQ + 8,), jnp.float32),
            pltpu.VMEM((_N,), jnp.float32),
        ],
        compiler_params=pltpu.CompilerParams(needs_layout_passes=False),
    )
    def k(x_hbm, out_hbm, x_v, o_v, srow):
        wid = lax.axis_index("s") * 2 + lax.axis_index("c")
        lane = lax.iota(jnp.int32, 16)
        mk9 = lane < _NQ
        qidx = ((lane + 1).astype(jnp.float32) * (0.1 * (_N - 1)) + 0.5).astype(jnp.int32)
        qidx = jnp.minimum(qidx, _N - 1)

        def block_body(blk, carry):
            start = wid * _RPW + blk * _RB
            pltpu.sync_copy(x_hbm.at[pl.ds(start * _N, _RB * _N)], x_v)

            def row_body(r, c2):
                base = r * _N
                a = x_v[pl.ds(base, 16)]
                b = x_v[pl.ds(base + 16, 16)]
                c = x_v[pl.ds(base + 32, 16)]
                d = x_v[pl.ds(base + 48, 16)]
                s0, s1, s2, s3 = _sort64(a, b, c, d)
                srow[pl.ds(0, 16)] = s0
                srow[pl.ds(16, 16)] = s1
                srow[pl.ds(32, 16)] = s2
                srow[pl.ds(48, 16)] = s3
                q = plsc.load_gather(srow, [qidx], mask=mk9)
                plsc.store_compressed(o_v.at[pl.ds(r * _NQ, 16)], q, mask=mk9)
                return c2

            lax.fori_loop(0, _RB, row_body, 0, unroll=2)
            pltpu.sync_copy(
                o_v.at[pl.ds(0, _RB * _NQ)],
                out_hbm.at[pl.ds(start * _NQ, _RB * _NQ)],
            )
            return carry

        lax.fori_loop(0, _NBLK, block_body, 0)

    return k


def kernel(x):
    b, t, n = x.shape
    xf = x.reshape(b * t * n)
    out = _make_kernel()(xf)
    return out.reshape(b, t, _NQ)


# speedup vs baseline: 1.0531x; 1.0531x over previous
import functools
import jax
import jax.numpy as jnp
from jax import lax
from jax.experimental import pallas as pl
from jax.experimental.pallas import tpu as pltpu
from jax.experimental.pallas import tpu_sc as plsc

_N = 64
_NQ = 9
_RB = 512
_NW = 32
_ROWS = 1024 * 512
_RPW = _ROWS // _NW
_NBLK = _RPW // _RB


def _sort_desc(v):
    return plsc.sort_key_val(v, v, descending=True)[0]


def _sort64(a, b, c, d):
    a = lax.sort(a)
    b = _sort_desc(b)
    c = lax.sort(c)
    d = _sort_desc(d)
    lo, hi = jnp.minimum(a, b), jnp.maximum(a, b)
    a2, b2 = lax.sort(lo), lax.sort(hi)
    lo, hi = jnp.minimum(c, d), jnp.maximum(c, d)
    c2, d2 = _sort_desc(hi), _sort_desc(lo)
    l0, l1 = jnp.minimum(a2, c2), jnp.minimum(b2, d2)
    h0, h1 = jnp.maximum(a2, c2), jnp.maximum(b2, d2)
    s0 = lax.sort(jnp.minimum(l0, l1))
    s1 = lax.sort(jnp.maximum(l0, l1))
    s2 = lax.sort(jnp.minimum(h0, h1))
    s3 = lax.sort(jnp.maximum(h0, h1))
    return s0, s1, s2, s3


def _make_kernel():
    mesh = plsc.VectorSubcoreMesh(core_axis_name="c", subcore_axis_name="s")

    @functools.partial(
        pl.kernel,
        mesh=mesh,
        out_type=jax.ShapeDtypeStruct((_ROWS * _NQ,), jnp.float32),
        scratch_types=[
            pltpu.VMEM((_RB * _N,), jnp.float32),
            pltpu.VMEM((_RB * _NQ + 8,), jnp.float32),
        ],
        compiler_params=pltpu.CompilerParams(needs_layout_passes=False),
    )
    def k(x_hbm, out_hbm, x_v, o_v):
        wid = lax.axis_index("s") * 2 + lax.axis_index("c")
        lane = lax.iota(jnp.int32, 16)
        m0 = (lane == 6) | (lane == 13)
        m1 = (lane == 3) | (lane == 9)
        m2 = (lane == 0) | (lane == 6) | (lane == 12)
        m3 = (lane == 2) | (lane == 9)
        i0 = jnp.where(lane == 13, 1, 0)
        i1 = jnp.where(lane == 3, 2, 3)
        i2 = jnp.where(lane == 0, 4, jnp.where(lane == 6, 5, 6))
        i3 = jnp.where(lane == 2, 7, 8)

        def block_body(blk, carry):
            start = wid * _RPW + blk * _RB
            pltpu.sync_copy(x_hbm.at[pl.ds(start * _N, _RB * _N)], x_v)

            def row_body(r, c2):
                base = r * _N
                a = x_v[pl.ds(base, 16)]
                b = x_v[pl.ds(base + 16, 16)]
                c = x_v[pl.ds(base + 32, 16)]
                d = x_v[pl.ds(base + 48, 16)]
                s0, s1, s2, s3 = _sort64(a, b, c, d)
                r9 = r * _NQ
                plsc.store_scatter(o_v, [i0 + r9], s0, mask=m0)
                plsc.store_scatter(o_v, [i1 + r9], s1, mask=m1)
                plsc.store_scatter(o_v, [i2 + r9], s2, mask=m2)
                plsc.store_scatter(o_v, [i3 + r9], s3, mask=m3)
                return c2

            lax.fori_loop(0, _RB, row_body, 0, unroll=4)
            pltpu.sync_copy(
                o_v.at[pl.ds(0, _RB * _NQ)],
                out_hbm.at[pl.ds(start * _NQ, _RB * _NQ)],
            )
            return carry

        lax.fori_loop(0, _NBLK, block_body, 0)

    return k


def kernel(x):
    b, t, n = x.shape
    xf = x.reshape(b * t * n)
    out = _make_kernel()(xf)
    return out.reshape(b, t, _NQ)


# speedup vs baseline: 1.6592x; 1.5756x over previous
import functools
import jax
import jax.numpy as jnp
from jax import lax
from jax.experimental import pallas as pl
from jax.experimental.pallas import tpu as pltpu
from jax.experimental.pallas import tpu_sc as plsc

_N = 64
_NQ = 9
_RB = 512
_NW = 32
_ROWS = 1024 * 512
_RPW = _ROWS // _NW
_NBLK = _RPW // _RB


def _sort_desc(v):
    return plsc.sort_key_val(v, v, descending=True)[0]


def _sort64(a, b, c, d):
    a = lax.sort(a)
    b = _sort_desc(b)
    c = lax.sort(c)
    d = _sort_desc(d)
    lo, hi = jnp.minimum(a, b), jnp.maximum(a, b)
    a2, b2 = lax.sort(lo), lax.sort(hi)
    lo, hi = jnp.minimum(c, d), jnp.maximum(c, d)
    c2, d2 = _sort_desc(hi), _sort_desc(lo)
    l0, l1 = jnp.minimum(a2, c2), jnp.minimum(b2, d2)
    h0, h1 = jnp.maximum(a2, c2), jnp.maximum(b2, d2)
    s0 = lax.sort(jnp.minimum(l0, l1))
    s1 = lax.sort(jnp.maximum(l0, l1))
    s2 = lax.sort(jnp.minimum(h0, h1))
    s3 = lax.sort(jnp.maximum(h0, h1))
    return s0, s1, s2, s3


def _make_kernel():
    mesh = plsc.VectorSubcoreMesh(core_axis_name="c", subcore_axis_name="s")

    @functools.partial(
        pl.kernel,
        mesh=mesh,
        out_type=jax.ShapeDtypeStruct((_ROWS * _NQ,), jnp.float32),
        scratch_types=[
            pltpu.VMEM((_RB * _N,), jnp.float32),
            pltpu.VMEM((_RB * _NQ + 8,), jnp.float32),
        ],
        compiler_params=pltpu.CompilerParams(needs_layout_passes=False),
    )
    def k(x_hbm, out_hbm, x_v, o_v):
        wid = lax.axis_index("s") * 2 + lax.axis_index("c")
        lane = lax.iota(jnp.int32, 16)
        m0 = (lane == 6) | (lane == 13)
        m1 = (lane == 3) | (lane == 9)
        m2 = (lane == 0) | (lane == 6) | (lane == 12)
        m3 = (lane == 2) | (lane == 9)
        i0 = jnp.where(lane == 13, 1, 0)
        i1 = jnp.where(lane == 3, 2, 3)
        i2 = jnp.where(lane == 0, 4, jnp.where(lane == 6, 5, 6))
        i3 = jnp.where(lane == 2, 7, 8)

        def block_body(blk, carry):
            start = wid * _RPW + blk * _RB
            pltpu.sync_copy(x_hbm.at[pl.ds(start * _N, _RB * _N)], x_v)

            @plsc.parallel_loop(0, _RB, 1, unroll=4)
            def row_body(r):
                base = r * _N
                a = x_v[pl.ds(base, 16)]
                b = x_v[pl.ds(base + 16, 16)]
                c = x_v[pl.ds(base + 32, 16)]
                d = x_v[pl.ds(base + 48, 16)]
                s0, s1, s2, s3 = _sort64(a, b, c, d)
                r9 = r * _NQ
                plsc.store_scatter(o_v, [i0 + r9], s0, mask=m0)
                plsc.store_scatter(o_v, [i1 + r9], s1, mask=m1)
                plsc.store_scatter(o_v, [i2 + r9], s2, mask=m2)
                plsc.store_scatter(o_v, [i3 + r9], s3, mask=m3)

            pltpu.sync_copy(
                o_v.at[pl.ds(0, _RB * _NQ)],
                out_hbm.at[pl.ds(start * _NQ, _RB * _NQ)],
            )
            return carry

        lax.fori_loop(0, _NBLK, block_body, 0)

    return k


def kernel(x):
    b, t, n = x.shape
    xf = x.reshape(b * t * n)
    out = _make_kernel()(xf)
    return out.reshape(b, t, _NQ)
